# Initial kernel scaffold; baseline (speedup 1.0000x reference)
import jax
import jax.numpy as jnp
from jax.experimental import pallas as pl


def kernel(position_ids, table):
    raise NotImplementedError("write your pallas kernel here")


# speedup vs baseline: 1.5499x; 1.5499x over previous
import functools

import jax
import jax.numpy as jnp
from jax import lax
from jax.experimental import pallas as pl
from jax.experimental.pallas import tpu as pltpu
from jax.experimental.pallas import tpu_sc as plsc

SEQ = 8192
DIM = 2048
NC, NS = 2, 16
NW = NC * NS
B = 4 * 8192
BPW = B // NW
K = 16
NBUF = 2
NCHUNK = BPW // K

_mesh = plsc.VectorSubcoreMesh(core_axis_name="c", subcore_axis_name="s")


@functools.partial(
    pl.kernel,
    out_type=jax.ShapeDtypeStruct((B, DIM), jnp.float32),
    mesh=_mesh,
    scratch_types=(
        [pltpu.VMEM((NCHUNK, K), jnp.int32)]
        + [pltpu.VMEM((K, DIM), jnp.float32) for _ in range(NBUF)]
        + [pltpu.SemaphoreType.DMA for _ in range(2 * NBUF)]
    ),
)
def _sc_gather(idx_hbm, table_hbm, out_hbm, idx_v, *rest):
    bufs = rest[:NBUF]
    gsems = rest[NBUF:2 * NBUF]
    wsems = rest[2 * NBUF:]
    wid = lax.axis_index("s") * NC + lax.axis_index("c")
    base = wid * BPW

    pltpu.sync_copy(idx_hbm.at[wid], idx_v)

    def g_start(c, b):
        pltpu.async_copy(table_hbm.at[idx_v.at[c]], bufs[b], gsems[b])

    def g_wait(b):
        pltpu.make_async_copy(table_hbm.at[idx_v.at[0]], bufs[b], gsems[b]).wait()

    def w_start(c, b):
        pltpu.async_copy(bufs[b], out_hbm.at[pl.ds(base + c * K, K)], wsems[b])

    def w_wait(b):
        pltpu.make_async_copy(bufs[b], out_hbm.at[pl.ds(base, K)], wsems[b]).wait()

    for b in range(NBUF):
        g_start(b, b)

    @pl.loop(0, NCHUNK - NBUF, step=NBUF)
    def _steady(g):
        for b in range(NBUF):
            g_wait(b)
            w_start(g + b, b)
        for b in range(NBUF):
            w_wait(b)
            g_start(g + b + NBUF, b)

    for b in range(NBUF):
        g_wait(b)
        w_start(NCHUNK - NBUF + b, b)
    for b in range(NBUF):
        w_wait(b)


def kernel(position_ids, table):
    idx = position_ids.astype(jnp.int32).reshape(NW, NCHUNK, K)
    out = _sc_gather(idx, table)
    return out.reshape(position_ids.shape + (DIM,))


# speedup vs baseline: 1.5720x; 1.0143x over previous
import functools

import jax
import jax.numpy as jnp
from jax import lax
from jax.experimental import pallas as pl
from jax.experimental.pallas import tpu as pltpu
from jax.experimental.pallas import tpu_sc as plsc

SEQ = 8192
DIM = 2048
NC, NS = 2, 16
NW = NC * NS
B = 4 * 8192
BPW = B // NW
K = 8
NBUF = 4
NCHUNK = BPW // K

_mesh = plsc.VectorSubcoreMesh(core_axis_name="c", subcore_axis_name="s")


@functools.partial(
    pl.kernel,
    out_type=jax.ShapeDtypeStruct((B, DIM), jnp.float32),
    mesh=_mesh,
    scratch_types=(
        [pltpu.VMEM((NCHUNK, K), jnp.int32)]
        + [pltpu.VMEM((K, DIM), jnp.float32) for _ in range(NBUF)]
        + [pltpu.SemaphoreType.DMA for _ in range(2 * NBUF)]
    ),
)
def _sc_gather(idx_hbm, table_hbm, out_hbm, idx_v, *rest):
    bufs = rest[:NBUF]
    gsems = rest[NBUF:2 * NBUF]
    wsems = rest[2 * NBUF:]
    wid = lax.axis_index("s") * NC + lax.axis_index("c")
    base = wid * BPW

    pltpu.sync_copy(idx_hbm.at[wid], idx_v)

    def g_start(c, b):
        pltpu.async_copy(table_hbm.at[idx_v.at[c]], bufs[b], gsems[b])

    def g_wait(b):
        pltpu.make_async_copy(table_hbm.at[idx_v.at[0]], bufs[b], gsems[b]).wait()

    def w_start(c, b):
        pltpu.async_copy(bufs[b], out_hbm.at[pl.ds(base + c * K, K)], wsems[b])

    def w_wait(b):
        pltpu.make_async_copy(bufs[b], out_hbm.at[pl.ds(base, K)], wsems[b]).wait()

    for b in range(NBUF):
        g_start(b, b)

    @pl.loop(0, NCHUNK - NBUF, step=NBUF)
    def _steady(g):
        for b in range(NBUF):
            g_wait(b)
            w_start(g + b, b)
        for b in range(NBUF):
            w_wait(b)
            g_start(g + b + NBUF, b)

    for b in range(NBUF):
        g_wait(b)
        w_start(NCHUNK - NBUF + b, b)
    for b in range(NBUF):
        w_wait(b)


def kernel(position_ids, table):
    idx = position_ids.astype(jnp.int32).reshape(NW, NCHUNK, K)
    out = _sc_gather(idx, table)
    return out.reshape(position_ids.shape + (DIM,))
